# Initial kernel scaffold; baseline (speedup 1.0000x reference)
import jax
import jax.numpy as jnp
from jax.experimental import pallas as pl


def kernel(x, edge_index, WQ_w, WQ_b, WK_w, WK_b, WV_w, WV_b, WO_w, WO_b, ln1_g, ln1_b, ln2_g, ln2_b, l1_w, l1_b, l2_w, l2_b):
    raise NotImplementedError("write your pallas kernel here")


# speedup vs baseline: 16.4122x; 16.4122x over previous
import functools

import jax
import jax.numpy as jnp
from jax import lax
from jax.experimental import pallas as pl
from jax.experimental.pallas import tpu as pltpu
from jax.experimental.pallas import tpu_sc as plsc

_N = 10000
_E = 320000
_D = 128
_H = 8
_DH = 16
_ACCW = 144
_B = 40
_NW = 32
_EW = _E // _NW
_NCHUNK = _EW // _B
_NP = 10240
_RPS = _NP // 16
_R = 1000
_SCALE = 0.25

_DN = (((1,), (1,)), ((), ()))
_GD = lax.GatherDimensionNumbers(offset_dims=(), collapsed_slice_dims=(0,),
                                 start_index_map=(0,))


def _qkv_body(x_ref, g_ref, b_ref, wq_ref, bq_ref, wk_ref, bk_ref,
              wv_ref, bv_ref, q_ref, kv_ref):
    x = x_ref[...]
    mu = jnp.mean(x, axis=1, keepdims=True)
    xc = x - mu
    var = jnp.mean(xc * xc, axis=1, keepdims=True)
    h = xc * lax.rsqrt(var + 1e-5) * g_ref[...] + b_ref[...]
    q_ref[...] = lax.dot_general(h, wq_ref[...], _DN,
                                 preferred_element_type=jnp.float32) + bq_ref[...]
    kv_ref[:, :_D] = lax.dot_general(h, wk_ref[...], _DN,
                                     preferred_element_type=jnp.float32) + bk_ref[...]
    kv_ref[:, _D:] = lax.dot_general(h, wv_ref[...], _DN,
                                     preferred_element_type=jnp.float32) + bv_ref[...]


def _edge_body(q_hbm, kv_hbm, src_hbm, dst_hbm, zero_hbm, out_hbm,
               src_v, dst_v, q_rows, kv_rows, msg, acc, sem_q, sem_kv):
    cid = lax.axis_index("c")
    sid = lax.axis_index("s")
    base0 = (cid * 16 + sid) * _EW
    pltpu.sync_copy(zero_hbm.at[pl.ds(sid * _RPS, _RPS)],
                    acc.at[pl.ds(sid * _RPS, _RPS)])
    plsc.subcore_barrier()
    lane = lax.iota(jnp.int32, 16)

    def chunk(i, carry):
        base = base0 + i * _B
        pltpu.sync_copy(src_hbm.at[pl.ds(base, _B)], src_v)
        pltpu.sync_copy(dst_hbm.at[pl.ds(base, _B)], dst_v)
        cq = pltpu.async_copy(q_hbm.at[dst_v], q_rows, sem_q)
        ck = pltpu.async_copy(kv_hbm.at[src_v], kv_rows, sem_kv)
        cq.wait()
        ck.wait()

        def edge(b, c2):
            den = jnp.zeros((16,), jnp.float32)
            for h in range(_H):
                qv = q_rows[b, pl.ds(h * _DH, _DH)]
                kk = kv_rows[b, pl.ds(h * _DH, _DH)]
                vv = kv_rows[b, pl.ds(_D + h * _DH, _DH)]
                s = qv * kk
                for r in (8, 4, 2, 1):
                    s = s + lax.gather(
                        s, ((lane + r) & 15).reshape(16, 1), _GD, (1,),
                        mode=lax.GatherScatterMode.PROMISE_IN_BOUNDS)
                ev = jnp.exp(s * _SCALE)
                msg[b, pl.ds(h * _DH, _DH)] = ev * vv
                den = den + jnp.where(lane == h, ev, 0.0)
            msg[b, pl.ds(_D, 16)] = den
            return c2

        lax.fori_loop(0, _B, edge, 0)
        pltpu.sync_copy(msg, acc.at[dst_v], add=True)
        return carry

    lax.fori_loop(0, _NCHUNK, chunk, 0)
    plsc.subcore_barrier()
    pltpu.sync_copy(acc.at[pl.ds(sid * _RPS, _RPS)],
                    out_hbm.at[pl.ds(cid * _NP + sid * _RPS, _RPS)])


@functools.cache
def _edge_call():
    return functools.partial(
        pl.kernel,
        out_type=jax.ShapeDtypeStruct((2 * _NP, _ACCW), jnp.float32),
        mesh=plsc.VectorSubcoreMesh(core_axis_name="c", subcore_axis_name="s"),
        compiler_params=pltpu.CompilerParams(use_tc_tiling_on_sc=False),
        scratch_types=[
            pltpu.VMEM((_B,), jnp.int32),
            pltpu.VMEM((_B,), jnp.int32),
            pltpu.VMEM((_B, _D), jnp.float32),
            pltpu.VMEM((_B, 2 * _D), jnp.float32),
            pltpu.VMEM((_B, _ACCW), jnp.float32),
            pltpu.VMEM_SHARED((_NP, _ACCW), jnp.float32),
            pltpu.SemaphoreType.DMA,
            pltpu.SemaphoreType.DMA,
        ],
    )(_edge_body)


def _final_body(acc0_ref, acc1_ref, x_ref, wo_ref, bo_ref, g2_ref, b2_ref,
                w1_ref, bf1_ref, w2_ref, bf2_ref, o_ref):
    a = acc0_ref[0] + acc1_ref[0]
    num = a[:, :_D]
    den = a[:, _D:_D + _H]
    hh = lax.broadcasted_iota(jnp.int32, (_H, _D), 0)
    dd = lax.broadcasted_iota(jnp.int32, (_H, _D), 1)
    sel = (dd // _DH == hh).astype(jnp.float32)
    den_full = lax.dot_general(den, sel, (((1,), (0,)), ((), ())),
                               preferred_element_type=jnp.float32)
    att = num / den_full
    o = lax.dot_general(att, wo_ref[...], _DN,
                        preferred_element_type=jnp.float32) + bo_ref[...]
    h = x_ref[...] + o
    mu = jnp.mean(h, axis=1, keepdims=True)
    hc = h - mu
    var = jnp.mean(hc * hc, axis=1, keepdims=True)
    hn = hc * lax.rsqrt(var + 1e-5) * g2_ref[...] + b2_ref[...]
    r = jnp.maximum(lax.dot_general(hn, w1_ref[...], _DN,
                                    preferred_element_type=jnp.float32) + bf1_ref[...], 0.0)
    m = lax.dot_general(r, w2_ref[...], _DN,
                        preferred_element_type=jnp.float32) + bf2_ref[...]
    o_ref[...] = h + m


def _row_spec(w):
    return pl.BlockSpec((_R, w), lambda i: (i, 0))


def _full_spec(shape):
    return pl.BlockSpec(shape, lambda i: tuple(0 for _ in shape))


_qkv_call = pl.pallas_call(
    _qkv_body,
    grid=(_N // _R,),
    in_specs=[
        _row_spec(_D),
        _full_spec((1, _D)), _full_spec((1, _D)),
        _full_spec((_D, _D)), _full_spec((1, _D)),
        _full_spec((_D, _D)), _full_spec((1, _D)),
        _full_spec((_D, _D)), _full_spec((1, _D)),
    ],
    out_specs=[_row_spec(_D), _row_spec(2 * _D)],
    out_shape=[
        jax.ShapeDtypeStruct((_N, _D), jnp.float32),
        jax.ShapeDtypeStruct((_N, 2 * _D), jnp.float32),
    ],
)

_final_call = pl.pallas_call(
    _final_body,
    grid=(_N // _R,),
    in_specs=[
        pl.BlockSpec((1, _R, _ACCW), lambda i: (0, i, 0)),
        pl.BlockSpec((1, _R, _ACCW), lambda i: (1, i, 0)),
        _row_spec(_D),
        _full_spec((_D, _D)), _full_spec((1, _D)),
        _full_spec((1, _D)), _full_spec((1, _D)),
        _full_spec((_D, _D)), _full_spec((1, _D)),
        _full_spec((_D, _D)), _full_spec((1, _D)),
    ],
    out_specs=_row_spec(_D),
    out_shape=jax.ShapeDtypeStruct((_N, _D), jnp.float32),
)


def kernel(x, edge_index, WQ_w, WQ_b, WK_w, WK_b, WV_w, WV_b, WO_w, WO_b,
           ln1_g, ln1_b, ln2_g, ln2_b, l1_w, l1_b, l2_w, l2_b):
    src = edge_index[0]
    dst = edge_index[1]
    row = lambda v: v.reshape(1, _D)
    q, kv = _qkv_call(x, row(ln1_g), row(ln1_b), WQ_w, row(WQ_b),
                      WK_w, row(WK_b), WV_w, row(WV_b))
    zeros = jnp.zeros((_NP, _ACCW), jnp.float32)
    acc = _edge_call()(q, kv, src, dst, zeros).reshape(2, _NP, _ACCW)
    return _final_call(acc, acc, x, WO_w, row(WO_b), row(ln2_g), row(ln2_b),
                       l1_w, row(l1_b), l2_w, row(l2_b))


# speedup vs baseline: 44.4324x; 2.7073x over previous
import functools

import jax
import jax.numpy as jnp
from jax import lax
from jax.experimental import pallas as pl
from jax.experimental.pallas import tpu as pltpu
from jax.experimental.pallas import tpu_sc as plsc

_N = 10000
_E = 320000
_D = 128
_H = 8
_DH = 16
_ACCW = 144
_B = 40
_NW = 32
_EW = _E // _NW
_NCHUNK = _EW // _B
_NP = 10240
_RPS = _NP // 16
_R = 1000
_SCALE = 0.25

_DN = (((1,), (1,)), ((), ()))
_GD = lax.GatherDimensionNumbers(offset_dims=(), collapsed_slice_dims=(0,),
                                 start_index_map=(0,))


def _qkv_body(x_ref, g_ref, b_ref, wq_ref, bq_ref, wk_ref, bk_ref,
              wv_ref, bv_ref, q_ref, kv_ref):
    x = x_ref[...]
    mu = jnp.mean(x, axis=1, keepdims=True)
    xc = x - mu
    var = jnp.mean(xc * xc, axis=1, keepdims=True)
    h = xc * lax.rsqrt(var + 1e-5) * g_ref[...] + b_ref[...]
    q_ref[...] = lax.dot_general(h, wq_ref[...], _DN,
                                 preferred_element_type=jnp.float32) + bq_ref[...]
    kv_ref[:, :_D] = lax.dot_general(h, wk_ref[...], _DN,
                                     preferred_element_type=jnp.float32) + bk_ref[...]
    kv_ref[:, _D:] = lax.dot_general(h, wv_ref[...], _DN,
                                     preferred_element_type=jnp.float32) + bv_ref[...]


def _edge_body(q_hbm, kv_hbm, src_hbm, dst_hbm, zero_hbm, out_hbm,
               src_v, dst_v, q_rows, kv_rows, msg, acc, sem_q, sem_kv):
    cid = lax.axis_index("c")
    sid = lax.axis_index("s")
    base0 = (cid * 16 + sid) * _EW
    pltpu.sync_copy(zero_hbm.at[pl.ds(sid * _RPS, _RPS)],
                    acc.at[pl.ds(sid * _RPS, _RPS)])
    plsc.subcore_barrier()
    lane = lax.iota(jnp.int32, 16)

    def chunk(i, carry):
        base = base0 + i * _B
        pltpu.sync_copy(src_hbm.at[pl.ds(base, _B)], src_v)
        pltpu.sync_copy(dst_hbm.at[pl.ds(base, _B)], dst_v)
        cq = pltpu.async_copy(q_hbm.at[dst_v], q_rows, sem_q)
        ck = pltpu.async_copy(kv_hbm.at[src_v], kv_rows, sem_kv)
        cq.wait()
        ck.wait()

        @plsc.parallel_loop(0, _B, unroll=4)
        def edge(b):
            den = jnp.zeros((16,), jnp.float32)
            for h in range(_H):
                qv = q_rows[b, pl.ds(h * _DH, _DH)]
                kk = kv_rows[b, pl.ds(h * _DH, _DH)]
                vv = kv_rows[b, pl.ds(_D + h * _DH, _DH)]
                s = qv * kk
                for r in (8, 4, 2, 1):
                    s = s + lax.gather(
                        s, ((lane + r) & 15).reshape(16, 1), _GD, (1,),
                        mode=lax.GatherScatterMode.PROMISE_IN_BOUNDS)
                ev = jnp.exp(s * _SCALE)
                msg[b, pl.ds(h * _DH, _DH)] = ev * vv
                den = den + jnp.where(lane == h, ev, 0.0)
            msg[b, pl.ds(_D, 16)] = den
        pltpu.sync_copy(msg, acc.at[dst_v], add=True)
        return carry

    lax.fori_loop(0, _NCHUNK, chunk, 0)
    plsc.subcore_barrier()
    pltpu.sync_copy(acc.at[pl.ds(sid * _RPS, _RPS)],
                    out_hbm.at[pl.ds(cid * _NP + sid * _RPS, _RPS)])


@functools.cache
def _edge_call():
    return functools.partial(
        pl.kernel,
        out_type=jax.ShapeDtypeStruct((2 * _NP, _ACCW), jnp.float32),
        mesh=plsc.VectorSubcoreMesh(core_axis_name="c", subcore_axis_name="s"),
        compiler_params=pltpu.CompilerParams(use_tc_tiling_on_sc=False),
        scratch_types=[
            pltpu.VMEM((_B,), jnp.int32),
            pltpu.VMEM((_B,), jnp.int32),
            pltpu.VMEM((_B, _D), jnp.float32),
            pltpu.VMEM((_B, 2 * _D), jnp.float32),
            pltpu.VMEM((_B, _ACCW), jnp.float32),
            pltpu.VMEM_SHARED((_NP, _ACCW), jnp.float32),
            pltpu.SemaphoreType.DMA,
            pltpu.SemaphoreType.DMA,
        ],
    )(_edge_body)


def _final_body(acc0_ref, acc1_ref, x_ref, wo_ref, bo_ref, g2_ref, b2_ref,
                w1_ref, bf1_ref, w2_ref, bf2_ref, o_ref):
    a = acc0_ref[0] + acc1_ref[0]
    num = a[:, :_D]
    den = a[:, _D:_D + _H]
    hh = lax.broadcasted_iota(jnp.int32, (_H, _D), 0)
    dd = lax.broadcasted_iota(jnp.int32, (_H, _D), 1)
    sel = (dd // _DH == hh).astype(jnp.float32)
    den_full = lax.dot_general(den, sel, (((1,), (0,)), ((), ())),
                               preferred_element_type=jnp.float32)
    att = num / den_full
    o = lax.dot_general(att, wo_ref[...], _DN,
                        preferred_element_type=jnp.float32) + bo_ref[...]
    h = x_ref[...] + o
    mu = jnp.mean(h, axis=1, keepdims=True)
    hc = h - mu
    var = jnp.mean(hc * hc, axis=1, keepdims=True)
    hn = hc * lax.rsqrt(var + 1e-5) * g2_ref[...] + b2_ref[...]
    r = jnp.maximum(lax.dot_general(hn, w1_ref[...], _DN,
                                    preferred_element_type=jnp.float32) + bf1_ref[...], 0.0)
    m = lax.dot_general(r, w2_ref[...], _DN,
                        preferred_element_type=jnp.float32) + bf2_ref[...]
    o_ref[...] = h + m


def _row_spec(w):
    return pl.BlockSpec((_R, w), lambda i: (i, 0))


def _full_spec(shape):
    return pl.BlockSpec(shape, lambda i: tuple(0 for _ in shape))


_qkv_call = pl.pallas_call(
    _qkv_body,
    grid=(_N // _R,),
    in_specs=[
        _row_spec(_D),
        _full_spec((1, _D)), _full_spec((1, _D)),
        _full_spec((_D, _D)), _full_spec((1, _D)),
        _full_spec((_D, _D)), _full_spec((1, _D)),
        _full_spec((_D, _D)), _full_spec((1, _D)),
    ],
    out_specs=[_row_spec(_D), _row_spec(2 * _D)],
    out_shape=[
        jax.ShapeDtypeStruct((_N, _D), jnp.float32),
        jax.ShapeDtypeStruct((_N, 2 * _D), jnp.float32),
    ],
)

_final_call = pl.pallas_call(
    _final_body,
    grid=(_N // _R,),
    in_specs=[
        pl.BlockSpec((1, _R, _ACCW), lambda i: (0, i, 0)),
        pl.BlockSpec((1, _R, _ACCW), lambda i: (1, i, 0)),
        _row_spec(_D),
        _full_spec((_D, _D)), _full_spec((1, _D)),
        _full_spec((1, _D)), _full_spec((1, _D)),
        _full_spec((_D, _D)), _full_spec((1, _D)),
        _full_spec((_D, _D)), _full_spec((1, _D)),
    ],
    out_specs=_row_spec(_D),
    out_shape=jax.ShapeDtypeStruct((_N, _D), jnp.float32),
)


def kernel(x, edge_index, WQ_w, WQ_b, WK_w, WK_b, WV_w, WV_b, WO_w, WO_b,
           ln1_g, ln1_b, ln2_g, ln2_b, l1_w, l1_b, l2_w, l2_b):
    src = edge_index[0]
    dst = edge_index[1]
    row = lambda v: v.reshape(1, _D)
    q, kv = _qkv_call(x, row(ln1_g), row(ln1_b), WQ_w, row(WQ_b),
                      WK_w, row(WK_b), WV_w, row(WV_b))
    zeros = jnp.zeros((_NP, _ACCW), jnp.float32)
    acc = _edge_call()(q, kv, src, dst, zeros).reshape(2, _NP, _ACCW)
    return _final_call(acc, acc, x, WO_w, row(WO_b), row(ln2_g), row(ln2_b),
                       l1_w, row(l1_b), l2_w, row(l2_b))
